# baseline (speedup 1.0000x reference)
import functools

import numpy as np

import jax
import jax.numpy as jnp
from jax.experimental import pallas as pl
from jax.experimental.pallas import tpu as pltpu

_TIME_RATIO = 0.5
_FREQ_RATIO = 0.4
_SEARCH_BITS = 26


def _build_dft_matrix(n_vars: int, patch_len: int) -> np.ndarray:
    V, L = n_vars, patch_len
    nh = L // 2
    l = np.arange(L)
    cols = []
    sum_blk = np.zeros((V * L, V), np.float32)
    for v in range(V):
        sum_blk[v * L:(v + 1) * L, v] = 1.0
    cols.append(sum_blk)
    for k in range(1, nh + 1):
        blk = np.zeros((V * L, V), np.float32)
        for v in range(V):
            blk[v * L:(v + 1) * L, v] = np.cos(2.0 * np.pi * k * l / L)
        cols.append(blk)
    for k in range(1, nh + 1):
        blk = np.zeros((V * L, V), np.float32)
        if k < nh:
            for v in range(V):
                blk[v * L:(v + 1) * L, v] = -np.sin(2.0 * np.pi * k * l / L)
        cols.append(blk)
    return np.concatenate(cols, axis=1)


def _build_ksum_matrices(n_vars: int, patch_len: int):
    V, nh = n_vars, patch_len // 2
    s1 = np.zeros((V * nh, V), np.float32)
    s2 = np.zeros((V * nh, V), np.float32)
    for k in range(nh):
        for v in range(V):
            s1[k * V + v, v] = 1.0
            s2[k * V + v, v] = 2.0 if k < nh - 1 else 1.0
    return s1, s2


def _build_expand_matrix(n_vars: int, patch_len: int) -> np.ndarray:
    V, L = n_vars, patch_len
    e = np.zeros((V, V * L), np.float32)
    for v in range(V):
        e[v, v * L:(v + 1) * L] = 1.0
    return e


def _split_bf16(a):
    hi = a.astype(jnp.bfloat16)
    lo = (a - hi.astype(jnp.float32)).astype(jnp.bfloat16)
    return hi, lo


def _sortable_u32(f):
    b = jax.lax.bitcast_convert_type(f, jnp.uint32)
    flip = jnp.where(b >= jnp.uint32(0x80000000),
                     jnp.uint32(0xFFFFFFFF), jnp.uint32(0x80000000))
    return b ^ flip


def _kth_threshold_lanes(u, k):
    cand = jnp.zeros(u.shape[:2] + (1,), jnp.uint32)
    for bit in range(31, 31 - _SEARCH_BITS, -1):
        trial = cand | jnp.uint32(1 << bit)
        cnt = jnp.sum((u >= trial).astype(jnp.int32), axis=2, keepdims=True)
        cand = jnp.where(cnt >= k, trial, cand)
    return cand


def _score_body(n_vars, patch_len, n_patch, b_per_blk, nblk, k_t, k_f,
                x_ref, d_ref, sk_ref, sk2_ref,
                cvt_ref, fnegt_ref, thrt_ref, thrf_ref,
                cvs_ref, fns_ref):
    V, L, P = n_vars, patch_len, n_patch
    nh = L // 2
    i = pl.program_id(0)
    x = x_ref[...]
    d = d_ref[...]
    x_hi, x_lo = _split_bf16(x)
    d_hi, d_lo = _split_bf16(d)
    dn = (((1,), (0,)), ((), ()))
    g = (jax.lax.dot_general(x_hi, d_hi, dn, preferred_element_type=jnp.float32)
         + (jax.lax.dot_general(x_hi, d_lo, dn, preferred_element_type=jnp.float32)
            + jax.lax.dot_general(x_lo, d_hi, dn, preferred_element_type=jnp.float32)))
    s1 = g[:, :V]
    gre = g[:, V:V + nh * V]
    gim = g[:, V + nh * V:V + 2 * nh * V]
    p2 = gre * gre + gim * gim
    mag = jnp.sqrt(p2)
    p2_hi, p2_lo = _split_bf16(p2)
    mag_hi, mag_lo = _split_bf16(mag)
    sk = sk_ref[...].astype(jnp.bfloat16)
    sk2 = sk2_ref[...].astype(jnp.bfloat16)
    fsum = (jax.lax.dot_general(mag_hi, sk, dn, preferred_element_type=jnp.float32)
            + jax.lax.dot_general(mag_lo, sk, dn, preferred_element_type=jnp.float32))
    sqs = (jax.lax.dot_general(p2_hi, sk2, dn, preferred_element_type=jnp.float32)
           + jax.lax.dot_general(p2_lo, sk2, dn, preferred_element_type=jnp.float32))
    var_s = jnp.maximum(sqs, 0.0)
    mean = s1 * (1.0 / L)
    cv = jnp.sqrt(var_s) / (mean + 1e-6)
    fneg = -(fsum + jnp.abs(s1))
    for j in range(b_per_blk):
        ct = jnp.transpose(cv[j * P:(j + 1) * P, :])
        ft_ = jnp.transpose(fneg[j * P:(j + 1) * P, :])
        cvt_ref[j] = ct
        fnegt_ref[j] = ft_
        cvs_ref[pl.ds(i * b_per_blk + j, 1)] = ct[None]
        fns_ref[pl.ds(i * b_per_blk + j, 1)] = ft_[None]

    @pl.when(i == nblk - 1)
    def _():
        thrt_ref[...] = _kth_threshold_lanes(_sortable_u32(cvs_ref[...]), k_t)
        thrf_ref[...] = _kth_threshold_lanes(_sortable_u32(fns_ref[...]), k_f)


def _apply_body(b_per_blk, n_patch, x_ref, cvt_ref, fnegt_ref, thrt_ref,
                thrf_ref, e_ref, tt_ref, ft_ref, o_ref):
    B, P = b_per_blk, n_patch
    u_t = _sortable_u32(cvt_ref[...])
    u_f = _sortable_u32(fnegt_ref[...])
    kt_t = jnp.where(u_t >= thrt_ref[...], 0.0, 1.0)
    kf_t = jnp.where(u_f >= thrf_ref[...], 0.0, 1.0)
    e = e_ref[...]
    dn0 = (((0,), (0,)), ((), ()))
    tt = tt_ref[0, 0]
    ft = ft_ref[0, 0]
    for j in range(B):
        a = jax.lax.dot_general(kt_t[j], e, dn0,
                                preferred_element_type=jnp.float32)
        b = jax.lax.dot_general(kf_t[j], e, dn0,
                                preferred_element_type=jnp.float32)
        sl = pl.ds(j * P, P)
        o_ref[sl, :] = 0.5 * (x_ref[sl, :] * (a + b)
                              + tt * (1.0 - a) + ft * (1.0 - b))


def kernel(x, time_mask_token, freq_mask_token):
    bs, P, V, L = x.shape
    nh = L // 2
    k_t = int(P * _TIME_RATIO)
    k_f = int(P * _FREQ_RATIO)
    rows = bs * P
    x2 = x.reshape(rows, V * L)
    d_mat = jnp.asarray(_build_dft_matrix(V, L))
    sk_np, sk2_np = _build_ksum_matrices(V, L)
    sk_mat = jnp.asarray(sk_np)
    sk2_mat = jnp.asarray(sk2_np)
    e_mat = jnp.asarray(_build_expand_matrix(V, L))
    dcols = d_mat.shape[1]

    BPB = 4
    R = BPB * P
    nblk = rows // R

    cvt, fnegt, thrt, thrf = pl.pallas_call(
        functools.partial(_score_body, V, L, P, BPB, nblk, k_t, k_f),
        grid=(nblk,),
        in_specs=[
            pl.BlockSpec((R, V * L), lambda i: (i, 0)),
            pl.BlockSpec((V * L, dcols), lambda i: (0, 0)),
            pl.BlockSpec((nh * V, V), lambda i: (0, 0)),
            pl.BlockSpec((nh * V, V), lambda i: (0, 0)),
        ],
        out_specs=[
            pl.BlockSpec((BPB, V, P), lambda i: (i, 0, 0)),
            pl.BlockSpec((BPB, V, P), lambda i: (i, 0, 0)),
            pl.BlockSpec((bs, V, 1), lambda i: (0, 0, 0)),
            pl.BlockSpec((bs, V, 1), lambda i: (0, 0, 0)),
        ],
        out_shape=[
            jax.ShapeDtypeStruct((bs, V, P), jnp.float32),
            jax.ShapeDtypeStruct((bs, V, P), jnp.float32),
            jax.ShapeDtypeStruct((bs, V, 1), jnp.uint32),
            jax.ShapeDtypeStruct((bs, V, 1), jnp.uint32),
        ],
        scratch_shapes=[
            pltpu.VMEM((bs, V, P), jnp.float32),
            pltpu.VMEM((bs, V, P), jnp.float32),
        ],
    )(x2, d_mat, sk_mat, sk2_mat)

    out2 = pl.pallas_call(
        functools.partial(_apply_body, BPB, P),
        grid=(nblk,),
        in_specs=[
            pl.BlockSpec((R, V * L), lambda i: (i, 0)),
            pl.BlockSpec((BPB, V, P), lambda i: (i, 0, 0)),
            pl.BlockSpec((BPB, V, P), lambda i: (i, 0, 0)),
            pl.BlockSpec((BPB, V, 1), lambda i: (i, 0, 0)),
            pl.BlockSpec((BPB, V, 1), lambda i: (i, 0, 0)),
            pl.BlockSpec((V, V * L), lambda i: (0, 0)),
            pl.BlockSpec((1, 1), lambda i: (0, 0)),
            pl.BlockSpec((1, 1), lambda i: (0, 0)),
        ],
        out_specs=pl.BlockSpec((R, V * L), lambda i: (i, 0)),
        out_shape=jax.ShapeDtypeStruct((rows, V * L), jnp.float32),
    )(x2, cvt, fnegt, thrt, thrf, e_mat,
      time_mask_token.reshape(1, 1), freq_mask_token.reshape(1, 1))

    return out2.reshape(bs, P, V, L)


# speedup vs baseline: 1.0788x; 1.0788x over previous
import functools

import numpy as np

import jax
import jax.numpy as jnp
from jax.experimental import pallas as pl
from jax.experimental.pallas import tpu as pltpu

_TIME_RATIO = 0.5
_FREQ_RATIO = 0.4
_SEARCH_BITS = 26


def _build_dft_matrix(n_vars: int, patch_len: int) -> np.ndarray:
    V, L = n_vars, patch_len
    nh = L // 2
    l = np.arange(L)
    cols = []
    sum_blk = np.zeros((V * L, V), np.float32)
    for v in range(V):
        sum_blk[v * L:(v + 1) * L, v] = 1.0
    cols.append(sum_blk)
    for k in range(1, nh + 1):
        blk = np.zeros((V * L, V), np.float32)
        for v in range(V):
            blk[v * L:(v + 1) * L, v] = np.cos(2.0 * np.pi * k * l / L)
        cols.append(blk)
    for k in range(1, nh + 1):
        blk = np.zeros((V * L, V), np.float32)
        if k < nh:
            for v in range(V):
                blk[v * L:(v + 1) * L, v] = -np.sin(2.0 * np.pi * k * l / L)
        cols.append(blk)
    return np.concatenate(cols, axis=1)


def _build_ksum_matrices(n_vars: int, patch_len: int):
    V, nh = n_vars, patch_len // 2
    s1 = np.zeros((V * nh, V), np.float32)
    s2 = np.zeros((V * nh, V), np.float32)
    for k in range(nh):
        for v in range(V):
            s1[k * V + v, v] = 1.0
            s2[k * V + v, v] = 2.0 if k < nh - 1 else 1.0
    return s1, s2


def _build_expand_matrix(n_vars: int, patch_len: int) -> np.ndarray:
    V, L = n_vars, patch_len
    e = np.zeros((V, V * L), np.float32)
    for v in range(V):
        e[v, v * L:(v + 1) * L] = 1.0
    return e


def _split_bf16(a):
    hi = a.astype(jnp.bfloat16)
    lo = (a - hi.astype(jnp.float32)).astype(jnp.bfloat16)
    return hi, lo


def _sortable_u32(f):
    b = jax.lax.bitcast_convert_type(f, jnp.uint32)
    flip = jnp.where(b >= jnp.uint32(0x80000000),
                     jnp.uint32(0xFFFFFFFF), jnp.uint32(0x80000000))
    return b ^ flip


def _kth_threshold_lanes(u, k):
    cand = jnp.zeros(u.shape[:2] + (1,), jnp.uint32)
    for bit in range(31, 31 - _SEARCH_BITS, -1):
        trial = cand | jnp.uint32(1 << bit)
        cnt = jnp.sum((u >= trial).astype(jnp.int32), axis=2, keepdims=True)
        cand = jnp.where(cnt >= k, trial, cand)
    return cand


def _score_body(n_vars, patch_len, n_patch, b_per_blk,
                x_ref, d_ref, sk_ref, sk2_ref,
                cvt_ref, fnegt_ref):
    V, L, P = n_vars, patch_len, n_patch
    nh = L // 2
    x = x_ref[...]
    d = d_ref[...]
    x_hi, x_lo = _split_bf16(x)
    d_hi, d_lo = _split_bf16(d)
    dn = (((1,), (0,)), ((), ()))
    g = (jax.lax.dot_general(x_hi, d_hi, dn, preferred_element_type=jnp.float32)
         + (jax.lax.dot_general(x_hi, d_lo, dn, preferred_element_type=jnp.float32)
            + jax.lax.dot_general(x_lo, d_hi, dn, preferred_element_type=jnp.float32)))
    s1 = g[:, :V]
    gre = g[:, V:V + nh * V]
    gim = g[:, V + nh * V:V + 2 * nh * V]
    p2 = gre * gre + gim * gim
    mag = jnp.sqrt(p2)
    p2_hi, p2_lo = _split_bf16(p2)
    mag_hi, mag_lo = _split_bf16(mag)
    sk = sk_ref[...].astype(jnp.bfloat16)
    sk2 = sk2_ref[...].astype(jnp.bfloat16)
    fsum = (jax.lax.dot_general(mag_hi, sk, dn, preferred_element_type=jnp.float32)
            + jax.lax.dot_general(mag_lo, sk, dn, preferred_element_type=jnp.float32))
    sqs = (jax.lax.dot_general(p2_hi, sk2, dn, preferred_element_type=jnp.float32)
           + jax.lax.dot_general(p2_lo, sk2, dn, preferred_element_type=jnp.float32))
    var_s = jnp.maximum(sqs, 0.0)
    mean = s1 * (1.0 / L)
    cv = jnp.sqrt(var_s) / (mean + 1e-6)
    fneg = -(fsum + jnp.abs(s1))
    for j in range(b_per_blk):
        cvt_ref[j] = jnp.transpose(cv[j * P:(j + 1) * P, :])
        fnegt_ref[j] = jnp.transpose(fneg[j * P:(j + 1) * P, :])


def _thresh_body(k_t, k_f, cvt_ref, fnegt_ref, tt_ref, tf_ref):
    tt_ref[...] = _kth_threshold_lanes(_sortable_u32(cvt_ref[...]), k_t)
    tf_ref[...] = _kth_threshold_lanes(_sortable_u32(fnegt_ref[...]), k_f)


def _apply_body(b_per_blk, n_patch, x_ref, cvt_ref, fnegt_ref, thrt_ref,
                thrf_ref, e_ref, tt_ref, ft_ref, o_ref):
    B, P = b_per_blk, n_patch
    u_t = _sortable_u32(cvt_ref[...])
    u_f = _sortable_u32(fnegt_ref[...])
    kt_t = jnp.where(u_t >= thrt_ref[...], 0.0, 1.0)
    kf_t = jnp.where(u_f >= thrf_ref[...], 0.0, 1.0)
    e = e_ref[...]
    dn0 = (((0,), (0,)), ((), ()))
    tt = tt_ref[0, 0]
    ft = ft_ref[0, 0]
    for j in range(B):
        a = jax.lax.dot_general(kt_t[j], e, dn0,
                                preferred_element_type=jnp.float32)
        b = jax.lax.dot_general(kf_t[j], e, dn0,
                                preferred_element_type=jnp.float32)
        sl = pl.ds(j * P, P)
        o_ref[sl, :] = 0.5 * (x_ref[sl, :] * (a + b)
                              + tt * (1.0 - a) + ft * (1.0 - b))


def kernel(x, time_mask_token, freq_mask_token):
    bs, P, V, L = x.shape
    nh = L // 2
    k_t = int(P * _TIME_RATIO)
    k_f = int(P * _FREQ_RATIO)
    rows = bs * P
    x2 = x.reshape(rows, V * L)
    d_mat = jnp.asarray(_build_dft_matrix(V, L))
    sk_np, sk2_np = _build_ksum_matrices(V, L)
    sk_mat = jnp.asarray(sk_np)
    sk2_mat = jnp.asarray(sk2_np)
    e_mat = jnp.asarray(_build_expand_matrix(V, L))
    dcols = d_mat.shape[1]

    BPB = 4
    R = BPB * P
    nblk = rows // R

    cvt, fnegt = pl.pallas_call(
        functools.partial(_score_body, V, L, P, BPB),
        grid=(nblk,),
        in_specs=[
            pl.BlockSpec((R, V * L), lambda i: (i, 0)),
            pl.BlockSpec((V * L, dcols), lambda i: (0, 0)),
            pl.BlockSpec((nh * V, V), lambda i: (0, 0)),
            pl.BlockSpec((nh * V, V), lambda i: (0, 0)),
        ],
        out_specs=[
            pl.BlockSpec((BPB, V, P), lambda i: (i, 0, 0)),
            pl.BlockSpec((BPB, V, P), lambda i: (i, 0, 0)),
        ],
        out_shape=[
            jax.ShapeDtypeStruct((bs, V, P), jnp.float32),
            jax.ShapeDtypeStruct((bs, V, P), jnp.float32),
        ],
    )(x2, d_mat, sk_mat, sk2_mat)

    thrt, thrf = pl.pallas_call(
        functools.partial(_thresh_body, k_t, k_f),
        in_specs=[
            pl.BlockSpec((bs, V, P), lambda: (0, 0, 0)),
            pl.BlockSpec((bs, V, P), lambda: (0, 0, 0)),
        ],
        out_specs=[
            pl.BlockSpec((bs, V, 1), lambda: (0, 0, 0)),
            pl.BlockSpec((bs, V, 1), lambda: (0, 0, 0)),
        ],
        out_shape=[
            jax.ShapeDtypeStruct((bs, V, 1), jnp.uint32),
            jax.ShapeDtypeStruct((bs, V, 1), jnp.uint32),
        ],
    )(cvt, fnegt)

    out2 = pl.pallas_call(
        functools.partial(_apply_body, BPB, P),
        grid=(nblk,),
        in_specs=[
            pl.BlockSpec((R, V * L), lambda i: (i, 0)),
            pl.BlockSpec((BPB, V, P), lambda i: (i, 0, 0)),
            pl.BlockSpec((BPB, V, P), lambda i: (i, 0, 0)),
            pl.BlockSpec((BPB, V, 1), lambda i: (i, 0, 0)),
            pl.BlockSpec((BPB, V, 1), lambda i: (i, 0, 0)),
            pl.BlockSpec((V, V * L), lambda i: (0, 0)),
            pl.BlockSpec((1, 1), lambda i: (0, 0)),
            pl.BlockSpec((1, 1), lambda i: (0, 0)),
        ],
        out_specs=pl.BlockSpec((R, V * L), lambda i: (i, 0)),
        out_shape=jax.ShapeDtypeStruct((rows, V * L), jnp.float32),
    )(x2, cvt, fnegt, thrt, thrf, e_mat,
      time_mask_token.reshape(1, 1), freq_mask_token.reshape(1, 1))

    return out2.reshape(bs, P, V, L)
